# baseline (speedup 1.0000x reference)
import functools

import jax
import jax.numpy as jnp
from jax import lax
from jax.experimental import pallas as pl
from jax.experimental.pallas import tpu as pltpu
from jax.experimental.pallas import tpu_sc as plsc

_NC = 2
_NS = 16
_L = 16
_NW = _NC * _NS
_LANES = 128
_SUBL = 8


def _reinterpret_noop(src_ref, dst_ref):
    del src_ref, dst_ref


def _flat_view(ent_t):
    d_model, n = ent_t.shape
    n_ti = -(-n // _LANES)
    padded = (d_model // _SUBL) * n_ti * _SUBL * _LANES
    return pl.pallas_call(
        _reinterpret_noop,
        out_shape=jax.ShapeDtypeStruct((padded,), ent_t.dtype),
        in_specs=[pl.BlockSpec(memory_space=pl.ANY)],
        out_specs=pl.BlockSpec(memory_space=pl.ANY),
        input_output_aliases={0: 0},
    )(ent_t)


def _make_body(num_edges, num_entities, d_model):
    n_ti = -(-num_entities // _LANES)
    slab = n_ti * _SUBL * _LANES
    padded = (d_model // _SUBL) * slab

    def body(edge_ref, ent_ref, rel_ref, out_ref,
             hidx, tidx, hbuf, tbuf, relv, outv, hsem, tsem):
        b_per_w = hidx.shape[0]
        wid = lax.axis_index("s") * _NC + lax.axis_index("c")
        base = wid * b_per_w

        pltpu.sync_copy(edge_ref.at[pl.ds(base, b_per_w)], hidx)
        pltpu.sync_copy(edge_ref.at[pl.ds(num_edges + base, b_per_w)], tidx)
        pltpu.sync_copy(rel_ref, relv)

        for d in range(d_model):
            src = ent_ref.at[d]
            dst = pl.ds(d * b_per_w, b_per_w)
            pltpu.async_copy(src.at[hidx], hbuf.at[dst], hsem)
            pltpu.async_copy(src.at[tidx], tbuf.at[dst], tsem)
        dummy = ent_ref.at[pl.ds(0, d_model * b_per_w)]
        pltpu.make_async_copy(dummy, hbuf, hsem).wait()
        pltpu.make_async_copy(dummy, tbuf, tsem).wait()

        r_parts = [relv[pl.ds(c * _L, _L)] for c in range(d_model // _L)]
        rscal = [r_parts[d // _L][d % _L] for d in range(d_model)]

        def group(g, carry):
            acc = jnp.zeros((_L,), jnp.float32)
            for d in range(d_model):
                sl = pl.ds(d * b_per_w + g * _L, _L)
                acc = acc + hbuf[sl] * tbuf[sl] * rscal[d]
            sig = 1.0 / (1.0 + jnp.exp(-acc))
            outv[pl.ds(g * _L, _L)] = sig
            return carry

        lax.fori_loop(0, b_per_w // _L, group, 0)
        pltpu.sync_copy(outv, out_ref.at[pl.ds(base, b_per_w)])

    return body


def kernel(edge_index, entity_emb, relation_emb):
    num_edges = edge_index.shape[1]
    num_entities, d_model = entity_emb.shape
    b_per_w = num_edges // _NW
    mesh = plsc.VectorSubcoreMesh(core_axis_name="c", subcore_axis_name="s")
    k = functools.partial(
        pl.kernel,
        mesh=mesh,
        out_type=jax.ShapeDtypeStruct((num_edges,), jnp.float32),
        compiler_params=pltpu.CompilerParams(
            needs_layout_passes=False, use_tc_tiling_on_sc=False),
        scratch_types=[
            pltpu.VMEM((b_per_w,), jnp.int32),
            pltpu.VMEM((b_per_w,), jnp.int32),
            pltpu.VMEM((b_per_w * d_model,), jnp.float32),
            pltpu.VMEM((b_per_w * d_model,), jnp.float32),
            pltpu.VMEM((d_model,), jnp.float32),
            pltpu.VMEM((b_per_w,), jnp.float32),
            pltpu.SemaphoreType.DMA,
            pltpu.SemaphoreType.DMA,
        ],
    )(_make_body(num_edges, num_entities, d_model))
    return k(edge_index.reshape(-1), entity_emb.T, relation_emb.reshape(-1))


# speedup vs baseline: 4.8882x; 4.8882x over previous
import functools

import jax
import jax.numpy as jnp
from jax import lax
from jax.experimental import pallas as pl
from jax.experimental.pallas import tpu as pltpu
from jax.experimental.pallas import tpu_sc as plsc

_NC = 2
_NS = 16
_L = 16
_NW = _NC * _NS


def _distmult_body(edge_ref, ent_ref, rel_ref, out_ref,
                   hidx, tidx, hrows, trows, relv, outv, hsem, tsem):
    b_per_w = hidx.shape[0]
    d_model = hrows.shape[1]
    wid = lax.axis_index("s") * _NC + lax.axis_index("c")
    base = wid * b_per_w

    pltpu.sync_copy(edge_ref.at[0, pl.ds(base, b_per_w)], hidx)
    pltpu.sync_copy(edge_ref.at[1, pl.ds(base, b_per_w)], tidx)
    pltpu.sync_copy(rel_ref.at[0], relv)
    hcopy = pltpu.async_copy(ent_ref.at[hidx], hrows, hsem)
    tcopy = pltpu.async_copy(ent_ref.at[tidx], trows, tsem)
    hcopy.wait()
    tcopy.wait()

    lanes = lax.iota(jnp.int32, _L)
    r_parts = [relv[pl.ds(c * _L, _L)] for c in range(d_model // _L)]
    rscal = [r_parts[d // _L][d % _L] for d in range(d_model)]

    def group(g, carry):
        eidx = g * _L + lanes
        acc = jnp.zeros((_L,), jnp.float32)
        for d in range(d_model):
            cd = jnp.full((_L,), d, jnp.int32)
            hcol = plsc.load_gather(hrows, [eidx, cd])
            tcol = plsc.load_gather(trows, [eidx, cd])
            acc = acc + hcol * tcol * rscal[d]
        sig = 1.0 / (1.0 + jnp.exp(-acc))
        outv[pl.ds(g * _L, _L)] = sig
        return carry

    lax.fori_loop(0, b_per_w // _L, group, 0)
    pltpu.sync_copy(outv, out_ref.at[pl.ds(base, b_per_w)])


def kernel(edge_index, entity_emb, relation_emb):
    num_edges = edge_index.shape[1]
    d_model = entity_emb.shape[1]
    b_per_w = num_edges // _NW
    mesh = plsc.VectorSubcoreMesh(core_axis_name="c", subcore_axis_name="s")
    k = functools.partial(
        pl.kernel,
        mesh=mesh,
        out_type=jax.ShapeDtypeStruct((num_edges,), jnp.float32),
        compiler_params=pltpu.CompilerParams(
            needs_layout_passes=False, use_tc_tiling_on_sc=False),
        scratch_types=[
            pltpu.VMEM((b_per_w,), jnp.int32),
            pltpu.VMEM((b_per_w,), jnp.int32),
            pltpu.VMEM((b_per_w, d_model), jnp.float32),
            pltpu.VMEM((b_per_w, d_model), jnp.float32),
            pltpu.VMEM((d_model,), jnp.float32),
            pltpu.VMEM((b_per_w,), jnp.float32),
            pltpu.SemaphoreType.DMA,
            pltpu.SemaphoreType.DMA,
        ],
    )(_distmult_body)
    return k(edge_index, entity_emb, relation_emb)


# speedup vs baseline: 4.9080x; 1.0040x over previous
import functools

import jax
import jax.numpy as jnp
from jax import lax
from jax.experimental import pallas as pl
from jax.experimental.pallas import tpu as pltpu
from jax.experimental.pallas import tpu_sc as plsc

_NC = 2
_NS = 16
_L = 16
_NW = _NC * _NS


def _distmult_body(edge_ref, ent_ref, rel_ref, out_ref,
                   hidx, tidx, hrows, trows, relv, outv, hsem, tsem):
    b_per_w = hidx.shape[0]
    d_model = hrows.shape[1]
    wid = lax.axis_index("s") * _NC + lax.axis_index("c")
    base = wid * b_per_w

    pltpu.sync_copy(edge_ref.at[0, pl.ds(base, b_per_w)], hidx)
    pltpu.sync_copy(edge_ref.at[1, pl.ds(base, b_per_w)], tidx)
    pltpu.sync_copy(rel_ref.at[0], relv)
    hcopy = pltpu.async_copy(ent_ref.at[hidx], hrows, hsem)
    tcopy = pltpu.async_copy(ent_ref.at[tidx], trows, tsem)
    hcopy.wait()
    tcopy.wait()

    lanes = lax.iota(jnp.int32, _L)
    r_parts = [relv[pl.ds(c * _L, _L)] for c in range(d_model // _L)]
    rscal = [r_parts[d // _L][d % _L] for d in range(d_model)]

    def group(g, carry):
        eidx = g * _L + lanes
        acc = jnp.zeros((_L,), jnp.float32)
        for d in range(d_model):
            cd = jnp.full((_L,), d, jnp.int32)
            hcol = plsc.load_gather(hrows, [eidx, cd])
            tcol = plsc.load_gather(trows, [eidx, cd])
            acc = acc + hcol * tcol * rscal[d]
        sig = 1.0 / (1.0 + jnp.exp(-acc))
        outv[pl.ds(g * _L, _L)] = sig
        return carry

    lax.fori_loop(0, b_per_w // _L, group, 0)
    pltpu.sync_copy(outv, out_ref.at[pl.ds(base, b_per_w)])


def kernel(edge_index, entity_emb, relation_emb):
    num_edges = edge_index.shape[1]
    d_model = entity_emb.shape[1]
    b_per_w = num_edges // _NW
    mesh = plsc.VectorSubcoreMesh(core_axis_name="c", subcore_axis_name="s")
    k = functools.partial(
        pl.kernel,
        mesh=mesh,
        out_type=jax.ShapeDtypeStruct((num_edges,), jnp.float32),
        compiler_params=pltpu.CompilerParams(
            needs_layout_passes=False, use_tc_tiling_on_sc=False,
            skip_device_barrier=True),
        scratch_types=[
            pltpu.VMEM((b_per_w,), jnp.int32),
            pltpu.VMEM((b_per_w,), jnp.int32),
            pltpu.VMEM((b_per_w, d_model), jnp.float32),
            pltpu.VMEM((b_per_w, d_model), jnp.float32),
            pltpu.VMEM((d_model,), jnp.float32),
            pltpu.VMEM((b_per_w,), jnp.float32),
            pltpu.SemaphoreType.DMA,
            pltpu.SemaphoreType.DMA,
        ],
    )(_distmult_body)
    return k(edge_index, entity_emb, relation_emb)
